# Initial kernel scaffold; baseline (speedup 1.0000x reference)
import jax
import jax.numpy as jnp
from jax.experimental import pallas as pl


def kernel(t, table, W1, b1, W2, b2):
    raise NotImplementedError("write your pallas kernel here")


# speedup vs baseline: 3.7538x; 3.7538x over previous
import functools

import jax
import jax.numpy as jnp
from jax import lax
from jax.experimental import pallas as pl
from jax.experimental.pallas import tpu as pltpu
from jax.experimental.pallas import tpu_sc as plsc

NUM_LABELS = 100000
D_MODEL = 64
DIM = 128


_ROW_BLK = 2000


def _mlp_body(tab_ref, w1_ref, b1_ref, w2_ref, b2_ref, out_ref):
    i = pl.program_id(0)
    x = tab_ref[...]
    row_ids = lax.broadcasted_iota(jnp.int32, (_ROW_BLK, 1), 0)
    x = jnp.where((row_ids == 0) & (i == 0), 0.0, x)
    h = jnp.dot(x, w1_ref[...], preferred_element_type=jnp.float32) + b1_ref[...]
    h = 0.5 * h * (1.0 + lax.erf(h * 0.7071067811865476))
    out_ref[...] = (
        jnp.dot(h, w2_ref[...], preferred_element_type=jnp.float32) + b2_ref[...]
    )


def _mlp_table(table, W1, b1, W2, b2):
    grid = (NUM_LABELS // _ROW_BLK,)
    return pl.pallas_call(
        _mlp_body,
        grid=grid,
        in_specs=[
            pl.BlockSpec((_ROW_BLK, D_MODEL), lambda i: (i, 0)),
            pl.BlockSpec((D_MODEL, DIM), lambda i: (0, 0)),
            pl.BlockSpec((1, DIM), lambda i: (0, 0)),
            pl.BlockSpec((DIM, DIM), lambda i: (0, 0)),
            pl.BlockSpec((1, DIM), lambda i: (0, 0)),
        ],
        out_specs=pl.BlockSpec((_ROW_BLK, DIM), lambda i: (i, 0)),
        out_shape=jax.ShapeDtypeStruct((NUM_LABELS, DIM), jnp.float32),
    )(table, W1, b1.reshape(1, DIM), W2, b2.reshape(1, DIM))


_NC = 2
_NS = 16
_NW = _NC * _NS
_N_TOK = 16384 * 50
_B_PER_W = _N_TOK // _NW
_CHUNK = 128
_N_CHUNKS = _B_PER_W // _CHUNK


@functools.cache
def _make_gather():
    mesh = plsc.VectorSubcoreMesh(core_axis_name="c", subcore_axis_name="s")

    @functools.partial(
        pl.kernel,
        mesh=mesh,
        out_type=jax.ShapeDtypeStruct((_N_TOK, DIM), jnp.float32),
        scratch_types=[
            pltpu.VMEM((_CHUNK,), jnp.int32),
            pltpu.VMEM((_CHUNK, DIM), jnp.float32),
            pltpu.SemaphoreType.DMA,
        ],
    )
    def gather_k(tab_hbm, idx_hbm, out_hbm, idx_v, rows_v, sem):
        wid = lax.axis_index("s") * _NC + lax.axis_index("c")
        base = wid * _B_PER_W

        def chunk(i, carry):
            off = base + i * _CHUNK
            pltpu.sync_copy(idx_hbm.at[pl.ds(off, _CHUNK)], idx_v)
            pltpu.async_copy(tab_hbm.at[idx_v], rows_v, sem).wait()
            pltpu.sync_copy(rows_v, out_hbm.at[pl.ds(off, _CHUNK)])
            return carry

        lax.fori_loop(0, _N_CHUNKS, chunk, 0)

    return gather_k


def kernel(t, table, W1, b1, W2, b2):
    B, L = t.shape
    mlp_tab = _mlp_table(table, W1, b1, W2, b2)
    out = _make_gather()(mlp_tab, t.reshape(-1))
    return out.reshape(B, L, DIM)


# speedup vs baseline: 4.3652x; 1.1629x over previous
import functools

import jax
import jax.numpy as jnp
from jax import lax
from jax.experimental import pallas as pl
from jax.experimental.pallas import tpu as pltpu
from jax.experimental.pallas import tpu_sc as plsc

NUM_LABELS = 100000
D_MODEL = 64
DIM = 128


_ROW_BLK = 2000


def _mlp_body(tab_ref, w1_ref, b1_ref, w2_ref, b2_ref, out_ref):
    i = pl.program_id(0)
    x = tab_ref[...]
    row_ids = lax.broadcasted_iota(jnp.int32, (_ROW_BLK, 1), 0)
    x = jnp.where((row_ids == 0) & (i == 0), 0.0, x)
    h = jnp.dot(x, w1_ref[...], preferred_element_type=jnp.float32) + b1_ref[...]
    h = 0.5 * h * (1.0 + lax.erf(h * 0.7071067811865476))
    out_ref[...] = (
        jnp.dot(h, w2_ref[...], preferred_element_type=jnp.float32) + b2_ref[...]
    )


def _mlp_table(table, W1, b1, W2, b2):
    grid = (NUM_LABELS // _ROW_BLK,)
    return pl.pallas_call(
        _mlp_body,
        grid=grid,
        in_specs=[
            pl.BlockSpec((_ROW_BLK, D_MODEL), lambda i: (i, 0)),
            pl.BlockSpec((D_MODEL, DIM), lambda i: (0, 0)),
            pl.BlockSpec((1, DIM), lambda i: (0, 0)),
            pl.BlockSpec((DIM, DIM), lambda i: (0, 0)),
            pl.BlockSpec((1, DIM), lambda i: (0, 0)),
        ],
        out_specs=pl.BlockSpec((_ROW_BLK, DIM), lambda i: (i, 0)),
        out_shape=jax.ShapeDtypeStruct((NUM_LABELS, DIM), jnp.float32),
    )(table, W1, b1.reshape(1, DIM), W2, b2.reshape(1, DIM))


_NC = 2
_NS = 16
_NW = _NC * _NS
_N_TOK = 16384 * 50
_B_PER_W = _N_TOK // _NW
_CHUNK = 80
_N_CHUNKS = _B_PER_W // _CHUNK
_NBUF = 8
_N_GROUPS = _N_CHUNKS // _NBUF


@functools.cache
def _make_gather():
    mesh = plsc.VectorSubcoreMesh(core_axis_name="c", subcore_axis_name="s")

    @functools.partial(
        pl.kernel,
        mesh=mesh,
        out_type=jax.ShapeDtypeStruct((_N_TOK, DIM), jnp.float32),
        scratch_types=[
            pltpu.VMEM((_B_PER_W,), jnp.int32),
            pltpu.VMEM((_NBUF, _CHUNK, DIM), jnp.float32),
            pltpu.SemaphoreType.DMA((_NBUF,)),
            pltpu.SemaphoreType.DMA((_NBUF,)),
        ],
    )
    def gather_k(tab_hbm, idx_hbm, out_hbm, idx_all, rows, gsem, wsem):
        wid = lax.axis_index("s") * _NC + lax.axis_index("c")
        base = wid * _B_PER_W
        pltpu.sync_copy(idx_hbm.at[pl.ds(base, _B_PER_W)], idx_all)

        def gcopy(b, i):
            src = tab_hbm.at[idx_all.at[pl.ds(i * _CHUNK, _CHUNK)]]
            return pltpu.make_async_copy(src, rows.at[b], gsem.at[b])

        def wcopy(b, i):
            dst = out_hbm.at[pl.ds(base + i * _CHUNK, _CHUNK)]
            return pltpu.make_async_copy(rows.at[b], dst, wsem.at[b])

        for b in range(4):
            gcopy(b, b).start()

        def group(k, carry):
            c0 = k * _NBUF
            for b in range(4):
                gcopy(b, c0 + b).wait()
            for b in range(4):
                wcopy(b, c0 + b).start()
            @pl.when(k > 0)
            def _():
                for b in range(4):
                    wcopy(4 + b, c0 - 4 + b).wait()
            for b in range(4):
                gcopy(4 + b, c0 + 4 + b).start()
            for b in range(4):
                gcopy(4 + b, c0 + 4 + b).wait()
            for b in range(4):
                wcopy(4 + b, c0 + 4 + b).start()
            for b in range(4):
                wcopy(b, c0 + b).wait()
            @pl.when(k + 1 < _N_GROUPS)
            def _():
                for b in range(4):
                    gcopy(b, c0 + _NBUF + b).start()
            return carry

        lax.fori_loop(0, _N_GROUPS, group, 0)
        for b in range(4):
            wcopy(4 + b, (_N_GROUPS - 1) * _NBUF + 4 + b).wait()

    return gather_k


def kernel(t, table, W1, b1, W2, b2):
    B, L = t.shape
    mlp_tab = _mlp_table(table, W1, b1, W2, b2)
    out = _make_gather()(mlp_tab, t.reshape(-1))
    return out.reshape(B, L, DIM)


# speedup vs baseline: 7.0167x; 1.6074x over previous
import functools

import jax
import jax.numpy as jnp
from jax import lax
from jax.experimental import pallas as pl
from jax.experimental.pallas import tpu as pltpu
from jax.experimental.pallas import tpu_sc as plsc

NUM_LABELS = 100000
D_MODEL = 64
DIM = 128


_ROW_BLK = 2000


def _mlp_body(tab_ref, w1_ref, b1_ref, w2_ref, b2_ref, out_ref):
    i = pl.program_id(0)
    x = tab_ref[...]
    row_ids = lax.broadcasted_iota(jnp.int32, (_ROW_BLK, 1), 0)
    x = jnp.where((row_ids == 0) & (i == 0), 0.0, x)
    h = jnp.dot(x, w1_ref[...], preferred_element_type=jnp.float32) + b1_ref[...]
    h = 0.5 * h * (1.0 + lax.erf(h * 0.7071067811865476))
    out_ref[...] = (
        jnp.dot(h, w2_ref[...], preferred_element_type=jnp.float32) + b2_ref[...]
    )


def _mlp_table(table, W1, b1, W2, b2):
    grid = (NUM_LABELS // _ROW_BLK,)
    return pl.pallas_call(
        _mlp_body,
        grid=grid,
        in_specs=[
            pl.BlockSpec((_ROW_BLK, D_MODEL), lambda i: (i, 0)),
            pl.BlockSpec((D_MODEL, DIM), lambda i: (0, 0)),
            pl.BlockSpec((1, DIM), lambda i: (0, 0)),
            pl.BlockSpec((DIM, DIM), lambda i: (0, 0)),
            pl.BlockSpec((1, DIM), lambda i: (0, 0)),
        ],
        out_specs=pl.BlockSpec((_ROW_BLK, DIM), lambda i: (i, 0)),
        out_shape=jax.ShapeDtypeStruct((NUM_LABELS, DIM), jnp.float32),
    )(table, W1, b1.reshape(1, DIM), W2, b2.reshape(1, DIM))


_NC = 2
_NS = 16
_NW = _NC * _NS
_B_SEQ = 16384
_L_SEQ = 50
_S_PER_W = _B_SEQ // _NW
_NBUF = 8
_N_GROUPS = _S_PER_W // _NBUF


@functools.cache
def _make_gather():
    mesh = plsc.VectorSubcoreMesh(core_axis_name="c", subcore_axis_name="s")

    @functools.partial(
        pl.kernel,
        mesh=mesh,
        out_type=jax.ShapeDtypeStruct((_B_SEQ, _L_SEQ, DIM), jnp.float32),
        scratch_types=[
            pltpu.VMEM((_S_PER_W, _L_SEQ), jnp.int32),
            pltpu.VMEM((_NBUF, _L_SEQ, DIM), jnp.float32),
            pltpu.SemaphoreType.DMA((_NBUF,)),
            pltpu.SemaphoreType.DMA((_NBUF,)),
        ],
    )
    def gather_k(tab_hbm, idx_hbm, out_hbm, idx_all, rows, gsem, wsem):
        wid = lax.axis_index("s") * _NC + lax.axis_index("c")
        base = wid * _S_PER_W
        pltpu.sync_copy(idx_hbm.at[pl.ds(base, _S_PER_W)], idx_all)

        def gcopy(b, i):
            src = tab_hbm.at[idx_all.at[i]]
            return pltpu.make_async_copy(src, rows.at[b], gsem.at[b])

        def wcopy(b, i):
            dst = out_hbm.at[base + i]
            return pltpu.make_async_copy(rows.at[b], dst, wsem.at[b])

        for b in range(4):
            gcopy(b, b).start()

        def group(k, carry):
            c0 = k * _NBUF
            for b in range(4):
                gcopy(b, c0 + b).wait()
            for b in range(4):
                wcopy(b, c0 + b).start()
            @pl.when(k > 0)
            def _():
                for b in range(4):
                    wcopy(4 + b, c0 - 4 + b).wait()
            for b in range(4):
                gcopy(4 + b, c0 + 4 + b).start()
            for b in range(4):
                gcopy(4 + b, c0 + 4 + b).wait()
            for b in range(4):
                wcopy(4 + b, c0 + 4 + b).start()
            for b in range(4):
                wcopy(b, c0 + b).wait()
            @pl.when(k + 1 < _N_GROUPS)
            def _():
                for b in range(4):
                    gcopy(b, c0 + _NBUF + b).start()
            return carry

        lax.fori_loop(0, _N_GROUPS, group, 0)
        for b in range(4):
            wcopy(4 + b, (_N_GROUPS - 1) * _NBUF + 4 + b).wait()

    return gather_k


def kernel(t, table, W1, b1, W2, b2):
    mlp_tab = _mlp_table(table, W1, b1, W2, b2)
    return _make_gather()(mlp_tab, t)
